# baseline (speedup 1.0000x reference)
import functools

import jax
import jax.numpy as jnp
from jax import lax
from jax.experimental import pallas as pl
from jax.experimental.pallas import tpu as pltpu
from jax.experimental.pallas import tpu_sc as plsc

_D_BYTE = 256
_D_FAM = 128
_D_MIC = 128
_DIM = _D_BYTE + _D_FAM + _D_MIC
_BATCH = 4
_SEQ = 8192
_B_TOTAL = _BATCH * _SEQ

_NC = 2
_NS = 16
_NW = _NC * _NS
_B_PER_W = _B_TOTAL // _NW
_CHUNK = 128
_N_CHUNKS = _B_PER_W // _CHUNK

_mesh = plsc.VectorSubcoreMesh(core_axis_name="c", subcore_axis_name="s")


@functools.partial(
    pl.kernel,
    mesh=_mesh,
    out_type=jax.ShapeDtypeStruct((_B_TOTAL, _DIM), jnp.float32),
    scratch_types=[
        pltpu.VMEM((_CHUNK,), jnp.int32),
        pltpu.VMEM((_CHUNK,), jnp.int32),
        pltpu.VMEM((_CHUNK,), jnp.int32),
        pltpu.VMEM((_CHUNK, _D_BYTE), jnp.float32),
        pltpu.VMEM((_CHUNK, _D_FAM), jnp.float32),
        pltpu.VMEM((_CHUNK, _D_MIC), jnp.float32),
        pltpu.SemaphoreType.DMA,
    ],
)
def _lookup_concat(ids_hbm, fam_hbm, mic_hbm, bt_hbm, ft_hbm, mt_hbm, out_hbm,
                   idx_b, idx_f, idx_m, rows_b, rows_f, rows_m, sem):
    wid = lax.axis_index("s") * _NC + lax.axis_index("c")
    base0 = wid * _B_PER_W

    def body(i, carry):
        base = base0 + i * _CHUNK
        pltpu.sync_copy(ids_hbm.at[pl.ds(base, _CHUNK)], idx_b)
        pltpu.sync_copy(fam_hbm.at[pl.ds(base, _CHUNK)], idx_f)
        pltpu.sync_copy(mic_hbm.at[pl.ds(base, _CHUNK)], idx_m)
        cb = pltpu.async_copy(bt_hbm.at[idx_b], rows_b, sem)
        cf = pltpu.async_copy(ft_hbm.at[idx_f], rows_f, sem)
        cm = pltpu.async_copy(mt_hbm.at[idx_m], rows_m, sem)
        cb.wait()
        cf.wait()
        cm.wait()
        pltpu.sync_copy(rows_b, out_hbm.at[pl.ds(base, _CHUNK), pl.ds(0, _D_BYTE)])
        pltpu.sync_copy(rows_f, out_hbm.at[pl.ds(base, _CHUNK), pl.ds(_D_BYTE, _D_FAM)])
        pltpu.sync_copy(rows_m, out_hbm.at[pl.ds(base, _CHUNK), pl.ds(_D_BYTE + _D_FAM, _D_MIC)])
        return carry

    lax.fori_loop(0, _N_CHUNKS, body, 0)


def kernel(input_ids, families, micro_refs, byte_table, family_table, micro_table):
    ids = input_ids.reshape(_B_TOTAL).astype(jnp.int32)
    fams = families.reshape(_B_TOTAL).astype(jnp.int32)
    mics = micro_refs.reshape(_B_TOTAL).astype(jnp.int32)
    out = _lookup_concat(ids, fams, mics, byte_table, family_table, micro_table)
    return out.reshape(_BATCH, _SEQ, _DIM)
